# baseline (speedup 1.0000x reference)
import functools

import jax
import jax.numpy as jnp
from jax import lax
from jax.experimental import pallas as pl
from jax.experimental.pallas import tpu as pltpu
from jax.experimental.pallas import tpu_sc as plsc

_D = 512
_BB = 512
_TT = 8


def _argmax_body(a_ref, o_ref):
    bb, tt, k = a_ref.shape
    x = a_ref[...].reshape(bb * tt, k)
    m = jnp.max(x, axis=1, keepdims=True)
    ii = jax.lax.broadcasted_iota(jnp.int32, x.shape, 1)
    idx = jnp.min(jnp.where(x == m, ii, k), axis=1, keepdims=True)
    idx = jnp.minimum(idx, k - 1)
    o_ref[...] = idx.reshape(1, 1, bb * tt).astype(jnp.int32)


def _argmax_idx(action, nj):
    b, t, k = action.shape
    grid = (b // _BB, nj)
    r = _BB * _TT
    return pl.pallas_call(
        _argmax_body,
        grid=grid,
        in_specs=[pl.BlockSpec((_BB, _TT, k), lambda i, j: (i, j, 0))],
        out_specs=pl.BlockSpec((1, 1, r), lambda i, j: (i * nj + j, 0, 0)),
        out_shape=jax.ShapeDtypeStruct(((b // _BB) * nj, 1, r), jnp.int32),
        compiler_params=pltpu.CompilerParams(
            dimension_semantics=("parallel", "parallel")),
    )(action)


def _sc_gather(table, idx, b, t, nj):
    info = plsc.get_sparse_core_info()
    nw = info.num_cores * info.num_subcores
    b_per_w = b // nw
    w_per_i = _BB // b_per_w
    lane_w = _BB * _TT // w_per_i
    t48 = nj * _TT
    nj_tot = t // _TT + 1
    mesh = plsc.VectorSubcoreMesh(core_axis_name="c", subcore_axis_name="s")

    @functools.partial(
        pl.kernel,
        out_type=jax.ShapeDtypeStruct((b, t, _D), jnp.float32),
        mesh=mesh,
        scratch_types=[
            pltpu.VMEM((nj * lane_w,), jnp.int32),
            pltpu.VMEM((t48, _D), jnp.float32),
            pltpu.SemaphoreType.DMA,
        ],
    )
    def k(table_hbm, idx_hbm, out_hbm, idx_v, rows_v, sem):
        wid = lax.axis_index("s") * info.num_cores + lax.axis_index("c")
        b0 = wid * b_per_w
        i = wid // w_per_i
        lane0 = (wid % w_per_i) * lane_w
        for j in range(nj):
            pltpu.sync_copy(idx_hbm.at[i * nj_tot + j, 0, pl.ds(lane0, lane_w)],
                            idx_v.at[pl.ds(j * lane_w, lane_w)])

        def body(bi, _):
            descs = [
                pltpu.async_copy(
                    table_hbm.at[idx_v.at[pl.ds(j * lane_w + bi * _TT, _TT)]],
                    rows_v.at[pl.ds(j * _TT, _TT)], sem)
                for j in range(nj)
            ]
            for d in descs:
                d.wait()
            pltpu.sync_copy(rows_v, out_hbm.at[b0 + bi, pl.ds(0, t48)])
            return ()

        lax.fori_loop(0, b_per_w, body, ())

    return k(table, idx)


def _tail_body(a_ref, w_ref, _, o_ref):
    bb, tt, k = a_ref.shape
    x = a_ref[...].reshape(bb * tt, k)
    m = jnp.max(x, axis=1, keepdims=True)
    ii = jax.lax.broadcasted_iota(jnp.int32, x.shape, 1)
    idx = jnp.min(jnp.where(x == m, ii, k), axis=1, keepdims=True)
    onehot = (ii == idx).astype(jnp.bfloat16)
    y = jax.lax.dot_general(
        onehot, w_ref[...], (((1,), (0,)), ((), ())),
        preferred_element_type=jnp.float32,
        precision=jax.lax.Precision.DEFAULT)
    o_ref[...] = y.reshape(bb, tt, _D)


def _tail(action, embed_weight, out_sc, jtail):
    b, t, k = action.shape
    return pl.pallas_call(
        _tail_body,
        grid=(b // _BB,),
        in_specs=[pl.BlockSpec((_BB, _TT, k), lambda i: (i, jtail, 0)),
                  pl.BlockSpec((k, _D), lambda i: (0, 0)),
                  pl.BlockSpec(memory_space=pl.ANY)],
        out_specs=pl.BlockSpec((_BB, _TT, _D), lambda i: (i, jtail, 0)),
        out_shape=jax.ShapeDtypeStruct((b, t, _D), jnp.float32),
        input_output_aliases={2: 0},
        compiler_params=pltpu.CompilerParams(
            dimension_semantics=("parallel",)),
    )(action, embed_weight, out_sc)


def kernel(action, embed_weight):
    b, t, k = action.shape
    nj_tot = pl.cdiv(t, _TT)
    nj_sc = t // _TT
    idx = _argmax_idx(action, nj_tot)
    out_sc = _sc_gather(embed_weight, idx, b, t, nj_sc)
    return _tail(action, embed_weight, out_sc, nj_sc)


# speedup vs baseline: 1.0154x; 1.0154x over previous
import functools

import jax
import jax.numpy as jnp
from jax import lax
from jax.experimental import pallas as pl
from jax.experimental.pallas import tpu as pltpu
from jax.experimental.pallas import tpu_sc as plsc

_D = 512
_BB = 512
_TT = 8


def _argmax_body(a_ref, o_ref):
    bb, tt, k = a_ref.shape
    x = a_ref[...].reshape(bb * tt, k)
    m = jnp.max(x, axis=1, keepdims=True)
    ii = jax.lax.broadcasted_iota(jnp.int32, x.shape, 1)
    idx = jnp.min(jnp.where(x == m, ii, k), axis=1, keepdims=True)
    idx = jnp.minimum(idx, k - 1)
    o_ref[...] = idx.reshape(1, 1, bb * tt).astype(jnp.int32)


def _argmax_idx(action, nj):
    b, t, k = action.shape
    grid = (b // _BB, nj)
    r = _BB * _TT
    return pl.pallas_call(
        _argmax_body,
        grid=grid,
        in_specs=[pl.BlockSpec((_BB, _TT, k), lambda i, j: (i, j, 0))],
        out_specs=pl.BlockSpec((1, 1, r), lambda i, j: (i * nj + j, 0, 0)),
        out_shape=jax.ShapeDtypeStruct(((b // _BB) * nj, 1, r), jnp.int32),
        compiler_params=pltpu.CompilerParams(
            dimension_semantics=("parallel", "parallel")),
    )(action)


def _sc_gather(table, idx, b, t, nj):
    info = plsc.get_sparse_core_info()
    nw = info.num_cores * info.num_subcores
    b_per_w = b // nw
    w_per_i = _BB // b_per_w
    lane_w = _BB * _TT // w_per_i
    t48 = nj * _TT
    nj_tot = t // _TT + 1
    mesh = plsc.VectorSubcoreMesh(core_axis_name="c", subcore_axis_name="s")

    nbuf = 4

    @functools.partial(
        pl.kernel,
        out_type=jax.ShapeDtypeStruct((b, t, _D), jnp.float32),
        mesh=mesh,
        scratch_types=(
            [pltpu.VMEM((nj * lane_w,), jnp.int32)]
            + [pltpu.VMEM((t48, _D), jnp.float32) for _ in range(nbuf)]
            + [pltpu.SemaphoreType.DMA for _ in range(2 * nbuf)]
        ),
    )
    def k(table_hbm, idx_hbm, out_hbm, idx_v, *bufs_and_sems):
        rows = bufs_and_sems[:nbuf]
        gsem = bufs_and_sems[nbuf:2 * nbuf]
        wsem = bufs_and_sems[2 * nbuf:]
        wid = lax.axis_index("s") * info.num_cores + lax.axis_index("c")
        b0 = wid * b_per_w
        i = wid // w_per_i
        lane0 = (wid % w_per_i) * lane_w
        for j in range(nj):
            pltpu.sync_copy(idx_hbm.at[i * nj_tot + j, 0, pl.ds(lane0, lane_w)],
                            idx_v.at[pl.ds(j * lane_w, lane_w)])

        def fire_g(bi, c):
            for j in range(nj):
                pltpu.async_copy(
                    table_hbm.at[idx_v.at[pl.ds(j * lane_w + bi * _TT, _TT)]],
                    rows[c].at[pl.ds(j * _TT, _TT)], gsem[c])

        def drain_g(c):
            for j in range(nj):
                pltpu.make_async_copy(
                    table_hbm.at[idx_v.at[pl.ds(j * lane_w, _TT)]],
                    rows[c].at[pl.ds(j * _TT, _TT)], gsem[c]).wait()

        def fire_w(bi, c):
            pltpu.async_copy(rows[c], out_hbm.at[b0 + bi, pl.ds(0, t48)],
                             wsem[c])

        def drain_w(c):
            pltpu.make_async_copy(rows[c], out_hbm.at[b0, pl.ds(0, t48)],
                                  wsem[c]).wait()

        for c in range(nbuf - 1):
            fire_g(c, c)

        nh = b_per_w // nbuf

        def body(h, _):
            for c in range(nbuf):
                bi = nbuf * h + c
                drain_g(c)
                fire_w(bi, c)
                d = (c + nbuf - 1) % nbuf

                @pl.when(bi + nbuf - 1 < b_per_w)
                def _():
                    @pl.when(bi - 1 >= 0)
                    def _():
                        drain_w(d)
                    fire_g(bi + nbuf - 1, d)
            return ()

        lax.fori_loop(0, nh, body, ())
        for c in range(nbuf):
            drain_w(c)

    return k(table, idx)


def _tail_body(a_ref, w_ref, _, o_ref):
    bb, tt, k = a_ref.shape
    x = a_ref[...].reshape(bb * tt, k)
    m = jnp.max(x, axis=1, keepdims=True)
    ii = jax.lax.broadcasted_iota(jnp.int32, x.shape, 1)
    idx = jnp.min(jnp.where(x == m, ii, k), axis=1, keepdims=True)
    onehot = (ii == idx).astype(jnp.bfloat16)
    y = jax.lax.dot_general(
        onehot, w_ref[...], (((1,), (0,)), ((), ())),
        preferred_element_type=jnp.float32,
        precision=jax.lax.Precision.DEFAULT)
    o_ref[...] = y.reshape(bb, tt, _D)


def _tail(action, embed_weight, out_sc, jtail):
    b, t, k = action.shape
    return pl.pallas_call(
        _tail_body,
        grid=(b // _BB,),
        in_specs=[pl.BlockSpec((_BB, _TT, k), lambda i: (i, jtail, 0)),
                  pl.BlockSpec((k, _D), lambda i: (0, 0)),
                  pl.BlockSpec(memory_space=pl.ANY)],
        out_specs=pl.BlockSpec((_BB, _TT, _D), lambda i: (i, jtail, 0)),
        out_shape=jax.ShapeDtypeStruct((b, t, _D), jnp.float32),
        input_output_aliases={2: 0},
        compiler_params=pltpu.CompilerParams(
            dimension_semantics=("parallel",)),
    )(action, embed_weight, out_sc)


def kernel(action, embed_weight):
    b, t, k = action.shape
    nj_tot = pl.cdiv(t, _TT)
    nj_sc = t // _TT
    idx = _argmax_idx(action, nj_tot)
    out_sc = _sc_gather(embed_weight, idx, b, t, nj_sc)
    return _tail(action, embed_weight, out_sc, nj_sc)


# speedup vs baseline: 7.0337x; 6.9273x over previous
import jax
import jax.numpy as jnp
from jax import lax
from jax.experimental import pallas as pl
from jax.experimental.pallas import tpu as pltpu

_D = 512
_TB = 2


def _body(a_ref, w_ref, o_ref):
    k = a_ref.shape[1]
    w = w_ref[...].astype(jnp.bfloat16)
    for i in range(a_ref.shape[0]):
        x = a_ref[i]
        m = jnp.max(x, axis=0, keepdims=True)
        ii = lax.broadcasted_iota(jnp.int32, x.shape, 0)
        idx = jnp.min(jnp.where(x == m, ii, k), axis=0, keepdims=True)
        onehot = (ii == idx).astype(jnp.bfloat16)
        o_ref[i] = lax.dot_general(
            onehot, w, (((0,), (0,)), ((), ())),
            preferred_element_type=jnp.float32,
            precision=jax.lax.Precision.DEFAULT)


def kernel(action, embed_weight):
    b, t, k = action.shape
    a_t = jnp.transpose(action, (1, 2, 0))
    out_t = pl.pallas_call(
        _body,
        grid=(t // _TB,),
        in_specs=[pl.BlockSpec((_TB, k, b), lambda tt: (tt, 0, 0)),
                  pl.BlockSpec((k, _D), lambda tt: (0, 0))],
        out_specs=pl.BlockSpec((_TB, b, _D), lambda tt: (tt, 0, 0)),
        out_shape=jax.ShapeDtypeStruct((t, b, _D), jnp.float32),
        compiler_params=pltpu.CompilerParams(
            dimension_semantics=("parallel",),
            vmem_limit_bytes=100 * 1024 * 1024),
    )(a_t, embed_weight)
    return jnp.transpose(out_t, (1, 0, 2))
